# Initial kernel scaffold; baseline (speedup 1.0000x reference)
import jax
import jax.numpy as jnp
from jax.experimental import pallas as pl


def kernel(coords, edge_index, atom_types, density_table, density_deriv_table, pair_table, pair_deriv_table, embed_table, embed_deriv_table):
    raise NotImplementedError("write your pallas kernel here")


# speedup vs baseline: 151.6261x; 151.6261x over previous
import jax
import jax.numpy as jnp
from jax import lax
from jax.experimental import pallas as pl
from jax.experimental.pallas import tpu as pltpu
from jax.experimental.pallas import tpu_sc as plsc

_NC = 2
_NS = 16
_NW = _NC * _NS
_L = 16
_CHUNK_A = 1024
_CHUNK_C = 512

_R_MAX = 6.0
_RHO_MAX = 50.0


def _rsqrt(s):
    i = lax.bitcast_convert_type(s, jnp.int32)
    i = jnp.int32(0x5F3759DF) - (i >> 1)
    y = lax.bitcast_convert_type(i, jnp.float32)
    half = jnp.float32(0.5) * s
    for _ in range(3):
        y = y * (jnp.float32(1.5) - half * y * y)
    return y


def _interp(tab_ref, b0, b1, frac):
    v0 = plsc.load_gather(tab_ref, [b0])
    v1 = plsc.load_gather(tab_ref, [b1])
    return v0 + frac * (v1 - v0)


def _r_to_table(r_vec, n_r):
    inv_dr = jnp.float32((n_r - 1) / _R_MAX)
    rc = jnp.minimum(r_vec, jnp.float32(_R_MAX * (1.0 - 1e-7)))
    rc = jnp.maximum(rc, jnp.float32(0.0))
    idxf = rc * inv_dr
    ii = idxf.astype(jnp.int32)
    fr = idxf - ii.astype(jnp.float32)
    ni = jnp.minimum(ii + 1, jnp.int32(n_r - 1))
    return ii, ni, fr


def _mesh():
    return plsc.VectorSubcoreMesh(core_axis_name="c", subcore_axis_name="s",
                                  num_cores=_NC, num_subcores=_NS)


def _sc_params():
    return pltpu.CompilerParams(use_tc_tiling_on_sc=False,
                                needs_layout_passes=False)


def kernel(coords, edge_index, atom_types, density_table, density_deriv_table,
           pair_table, pair_deriv_table, embed_table, embed_deriv_table):
    N = coords.shape[0]
    E = edge_index.shape[1]
    T, N_R = density_table.shape
    N_RHO = embed_table.shape[1]

    NPAD = -(-(N + 1) // 256) * 256
    EW = -(-(-(-E // _NW)) // _CHUNK_A) * _CHUNK_A
    EPAD = EW * _NW
    SL = NPAD // _NS
    AW = NPAD // _NW

    f32 = jnp.float32
    coords = coords.astype(f32)
    types_i = atom_types.astype(jnp.int32)

    tflt = types_i.astype(f32)
    packed_a = jnp.pad(jnp.concatenate([coords, tflt[:, None]], axis=1),
                       ((0, NPAD - N), (0, 12)))
    esrc = jnp.pad(edge_index[0].astype(jnp.int32), (0, EPAD - E),
                   constant_values=N).reshape(EPAD // 128, 128)
    edst = jnp.pad(edge_index[1].astype(jnp.int32), (0, EPAD - E),
                   constant_values=N).reshape(EPAD // 128, 128)
    tpad = jnp.pad(types_i, (0, NPAD - N))
    zeros_n = jnp.zeros((NPAD,), f32)
    dflat = density_table.reshape(-1).astype(f32)
    ddflat = density_deriv_table.reshape(-1).astype(f32)
    pflat = pair_table.reshape(-1).astype(f32)
    pdflat = pair_deriv_table.reshape(-1).astype(f32)
    eflat = embed_table.reshape(-1).astype(f32)
    edflat = embed_deriv_table.reshape(-1).astype(f32)

    def a_body(esrc_h, edst_h, atoms_h, dtab_h, zeros_h, rho_out,
               idx_s, idx_d, rows_s, rows_d, vals, dtab, rho_sp, sem):
        c0 = jnp.zeros((_L,), jnp.int32)
        c1 = jnp.full((_L,), 1, jnp.int32)
        c2 = jnp.full((_L,), 2, jnp.int32)
        c3 = jnp.full((_L,), 3, jnp.int32)
        c = lax.axis_index("c")
        s = lax.axis_index("s")
        wid = s * _NC + c
        crows = _CHUNK_A // 128
        pltpu.sync_copy(dtab_h, dtab)
        pltpu.sync_copy(zeros_h.at[pl.ds(s * SL, SL)],
                        rho_sp.at[pl.ds(s * SL, SL)])
        plsc.subcore_barrier()

        def chunk(j, carry):
            row0 = wid * (EW // 128) + j * crows
            pltpu.sync_copy(esrc_h.at[pl.ds(row0, crows)], idx_s)
            pltpu.sync_copy(edst_h.at[pl.ds(row0, crows)], idx_d)
            cps = []
            for r in range(crows):
                cps.append(pltpu.async_copy(atoms_h.at[idx_s.at[r]],
                                            rows_s.at[r], sem))
                cps.append(pltpu.async_copy(atoms_h.at[idx_d.at[r]],
                                            rows_d.at[r], sem))
            for cp in cps:
                cp.wait()

            def vec(q, acc):
                rv = jnp.full((_L,), q // 8, jnp.int32)
                lv = (q % 8) * _L + lax.iota(jnp.int32, _L)
                xs = plsc.load_gather(rows_s, [rv, lv, c0])
                ys = plsc.load_gather(rows_s, [rv, lv, c1])
                zs = plsc.load_gather(rows_s, [rv, lv, c2])
                tb = plsc.load_gather(rows_s, [rv, lv, c3])
                xd = plsc.load_gather(rows_d, [rv, lv, c0])
                yd = plsc.load_gather(rows_d, [rv, lv, c1])
                zd = plsc.load_gather(rows_d, [rv, lv, c2])
                dx = xd - xs
                dy = yd - ys
                dz = zd - zs
                s2 = dx * dx + dy * dy + dz * dz + jnp.float32(1e-12)
                rr = s2 * _rsqrt(s2)
                ii, ni, fr = _r_to_table(rr, N_R)
                ts = tb.astype(jnp.int32)
                b = ts * N_R
                vals[pl.ds(q * _L, _L)] = _interp(dtab, b + ii, b + ni, fr)
                return acc
            lax.fori_loop(0, _CHUNK_A // _L, vec, 0)
            for r in range(crows):
                pltpu.sync_copy(vals.at[pl.ds(r * 128, 128)],
                                rho_sp.at[idx_d.at[r]], add=True)
            return carry
        lax.fori_loop(0, EW // _CHUNK_A, chunk, 0)
        plsc.subcore_barrier()
        pltpu.sync_copy(rho_sp.at[pl.ds(s * SL, SL)],
                        rho_out.at[pl.ds(c * NPAD + s * SL, SL)])

    rho_part = pl.kernel(
        a_body,
        out_type=jax.ShapeDtypeStruct((_NC * NPAD,), f32),
        mesh=_mesh(),
        compiler_params=_sc_params(),
        scratch_types=[
            pltpu.VMEM((_CHUNK_A // 128, 128), jnp.int32),
            pltpu.VMEM((_CHUNK_A // 128, 128), jnp.int32),
            pltpu.VMEM((_CHUNK_A // 128, 128, 16), f32),
            pltpu.VMEM((_CHUNK_A // 128, 128, 16), f32),
            pltpu.VMEM((_CHUNK_A,), f32),
            pltpu.VMEM((T * N_R,), f32),
            pltpu.VMEM_SHARED((NPAD,), f32),
            pltpu.SemaphoreType.DMA,
        ],
    )(esrc, edst, packed_a, dflat, zeros_n)

    def b_body(rho_h, t_h, et_h, edt_h, fp_out, fsum_out,
               rho0, rho1, tv, fpv, et, edt, accbuf):
        c = lax.axis_index("c")
        s = lax.axis_index("s")
        wid = s * _NC + c
        base = wid * AW
        pltpu.sync_copy(et_h, et)
        pltpu.sync_copy(edt_h, edt)
        pltpu.sync_copy(rho_h.at[pl.ds(base, AW)], rho0)
        pltpu.sync_copy(rho_h.at[pl.ds(NPAD + base, AW)], rho1)
        pltpu.sync_copy(t_h.at[pl.ds(base, AW)], tv)
        inv_drho = jnp.float32((N_RHO - 1) / _RHO_MAX)

        def vec(q, acc):
            rho = rho0[pl.ds(q * _L, _L)] + rho1[pl.ds(q * _L, _L)]
            rc = jnp.minimum(jnp.maximum(rho, jnp.float32(0.0)),
                             jnp.float32(_RHO_MAX * (1.0 - 1e-7)))
            rf = rc * inv_drho
            ri = rf.astype(jnp.int32)
            fr = rf - ri.astype(jnp.float32)
            ni = jnp.minimum(ri + 1, jnp.int32(N_RHO - 1))
            eb = tv[pl.ds(q * _L, _L)] * N_RHO
            F = _interp(et, eb + ri, eb + ni, fr)
            Fp = _interp(edt, eb + ri, eb + ni, fr)
            fpv[pl.ds(q * _L, _L)] = Fp
            gidx = base + q * _L + lax.iota(jnp.int32, _L)
            return acc + jnp.where(gidx < N, F, jnp.float32(0.0))
        acc = lax.fori_loop(0, AW // _L, vec, jnp.zeros((_L,), f32))
        accbuf[...] = acc
        pltpu.sync_copy(fpv, fp_out.at[pl.ds(base, AW)])
        pltpu.sync_copy(accbuf, fsum_out.at[pl.ds(wid * _L, _L)])

    fp, fsum = pl.kernel(
        b_body,
        out_type=(jax.ShapeDtypeStruct((NPAD,), f32),
                  jax.ShapeDtypeStruct((_NW * _L,), f32)),
        mesh=_mesh(),
        compiler_params=_sc_params(),
        scratch_types=[
            pltpu.VMEM((AW,), f32),
            pltpu.VMEM((AW,), f32),
            pltpu.VMEM((AW,), jnp.int32),
            pltpu.VMEM((AW,), f32),
            pltpu.VMEM((T * N_RHO,), f32),
            pltpu.VMEM((T * N_RHO,), f32),
            pltpu.VMEM((_L,), f32),
        ],
    )(rho_part, tpad, eflat, edflat)

    packed_c = jnp.concatenate(
        [packed_a[:, :4], fp[:, None], jnp.zeros((NPAD, 11), f32)], axis=1)

    def c_body(esrc_h, edst_h, atoms_h, ddtab_h, ptab_h, pdtab_h, zeros_h,
               fx_out, fy_out, fz_out, psum_out,
               idx_s, idx_d, rows_s, rows_d, fxv, fyv, fzv,
               ddtab, ptab, pdtab, accbuf, fx_sp, fy_sp, fz_sp, sem):
        c0 = jnp.zeros((_L,), jnp.int32)
        c1 = jnp.full((_L,), 1, jnp.int32)
        c2 = jnp.full((_L,), 2, jnp.int32)
        c3 = jnp.full((_L,), 3, jnp.int32)
        c4 = jnp.full((_L,), 4, jnp.int32)
        c = lax.axis_index("c")
        s = lax.axis_index("s")
        wid = s * _NC + c
        crows = _CHUNK_C // 128
        pltpu.sync_copy(ddtab_h, ddtab)
        pltpu.sync_copy(ptab_h, ptab)
        pltpu.sync_copy(pdtab_h, pdtab)
        for t in (fx_sp, fy_sp, fz_sp):
            pltpu.sync_copy(zeros_h.at[pl.ds(s * SL, SL)],
                            t.at[pl.ds(s * SL, SL)])
        plsc.subcore_barrier()

        def chunk(j, acc):
            row0 = wid * (EW // 128) + j * crows
            pltpu.sync_copy(esrc_h.at[pl.ds(row0, crows)], idx_s)
            pltpu.sync_copy(edst_h.at[pl.ds(row0, crows)], idx_d)
            cps = []
            for r in range(crows):
                cps.append(pltpu.async_copy(atoms_h.at[idx_s.at[r]],
                                            rows_s.at[r], sem))
                cps.append(pltpu.async_copy(atoms_h.at[idx_d.at[r]],
                                            rows_d.at[r], sem))
            for cp in cps:
                cp.wait()
            ebase = wid * EW + j * _CHUNK_C

            def vec(q, a):
                rv = jnp.full((_L,), q // 8, jnp.int32)
                lv = (q % 8) * _L + lax.iota(jnp.int32, _L)
                xs = plsc.load_gather(rows_s, [rv, lv, c0])
                ys = plsc.load_gather(rows_s, [rv, lv, c1])
                zs = plsc.load_gather(rows_s, [rv, lv, c2])
                tbs = plsc.load_gather(rows_s, [rv, lv, c3])
                fps = plsc.load_gather(rows_s, [rv, lv, c4])
                xd = plsc.load_gather(rows_d, [rv, lv, c0])
                yd = plsc.load_gather(rows_d, [rv, lv, c1])
                zd = plsc.load_gather(rows_d, [rv, lv, c2])
                tbd = plsc.load_gather(rows_d, [rv, lv, c3])
                fpd = plsc.load_gather(rows_d, [rv, lv, c4])
                dx = xd - xs
                dy = yd - ys
                dz = zd - zs
                s2 = dx * dx + dy * dy + dz * dz + jnp.float32(1e-12)
                invr = _rsqrt(s2)
                rr = s2 * invr
                ii, ni, fr = _r_to_table(rr, N_R)
                ts = tbs.astype(jnp.int32)
                td = tbd.astype(jnp.int32)
                bs = ts * N_R
                bd = td * N_R
                rds = _interp(ddtab, bs + ii, bs + ni, fr)
                rdd = _interp(ddtab, bd + ii, bd + ni, fr)
                pb = (td * T + ts) * N_R
                phi = _interp(ptab, pb + ii, pb + ni, fr)
                phip = _interp(pdtab, pb + ii, pb + ni, fr)
                dEdr = fpd * rds + fps * rdd + phip
                coef = -dEdr * invr
                fxv[pl.ds(q * _L, _L)] = coef * dx
                fyv[pl.ds(q * _L, _L)] = coef * dy
                fzv[pl.ds(q * _L, _L)] = coef * dz
                gidx = ebase + q * _L + lax.iota(jnp.int32, _L)
                return a + jnp.where(gidx < E, phi, jnp.float32(0.0))
            acc = lax.fori_loop(0, _CHUNK_C // _L, vec, acc)
            for r in range(crows):
                pltpu.sync_copy(fxv.at[pl.ds(r * 128, 128)],
                                fx_sp.at[idx_d.at[r]], add=True)
                pltpu.sync_copy(fyv.at[pl.ds(r * 128, 128)],
                                fy_sp.at[idx_d.at[r]], add=True)
                pltpu.sync_copy(fzv.at[pl.ds(r * 128, 128)],
                                fz_sp.at[idx_d.at[r]], add=True)
            return acc
        acc = lax.fori_loop(0, EW // _CHUNK_C, chunk, jnp.zeros((_L,), f32))
        accbuf[...] = acc
        plsc.subcore_barrier()
        pltpu.sync_copy(fx_sp.at[pl.ds(s * SL, SL)],
                        fx_out.at[pl.ds(c * NPAD + s * SL, SL)])
        pltpu.sync_copy(fy_sp.at[pl.ds(s * SL, SL)],
                        fy_out.at[pl.ds(c * NPAD + s * SL, SL)])
        pltpu.sync_copy(fz_sp.at[pl.ds(s * SL, SL)],
                        fz_out.at[pl.ds(c * NPAD + s * SL, SL)])
        pltpu.sync_copy(accbuf, psum_out.at[pl.ds(wid * _L, _L)])

    fx_p, fy_p, fz_p, psum = pl.kernel(
        c_body,
        out_type=(jax.ShapeDtypeStruct((_NC * NPAD,), f32),
                  jax.ShapeDtypeStruct((_NC * NPAD,), f32),
                  jax.ShapeDtypeStruct((_NC * NPAD,), f32),
                  jax.ShapeDtypeStruct((_NW * _L,), f32)),
        mesh=_mesh(),
        compiler_params=_sc_params(),
        scratch_types=[
            pltpu.VMEM((_CHUNK_C // 128, 128), jnp.int32),
            pltpu.VMEM((_CHUNK_C // 128, 128), jnp.int32),
            pltpu.VMEM((_CHUNK_C // 128, 128, 16), f32),
            pltpu.VMEM((_CHUNK_C // 128, 128, 16), f32),
            pltpu.VMEM((_CHUNK_C,), f32),
            pltpu.VMEM((_CHUNK_C,), f32),
            pltpu.VMEM((_CHUNK_C,), f32),
            pltpu.VMEM((T * N_R,), f32),
            pltpu.VMEM((T * T * N_R,), f32),
            pltpu.VMEM((T * T * N_R,), f32),
            pltpu.VMEM((_L,), f32),
            pltpu.VMEM_SHARED((NPAD,), f32),
            pltpu.VMEM_SHARED((NPAD,), f32),
            pltpu.VMEM_SHARED((NPAD,), f32),
            pltpu.SemaphoreType.DMA,
        ],
    )(esrc, edst, packed_c, ddflat, pflat, pdflat, zeros_n)

    def d_body(fx_r, fy_r, fz_r, fs_r, ps_r, of_r, oe_r):
        of_r[0:1, :] = fx_r[0:1, :] + fx_r[1:2, :]
        of_r[1:2, :] = fy_r[0:1, :] + fy_r[1:2, :]
        of_r[2:3, :] = fz_r[0:1, :] + fz_r[1:2, :]
        e = jnp.sum(fs_r[...]) + jnp.float32(0.5) * jnp.sum(ps_r[...])
        oe_r[...] = jnp.reshape(e, (1, 1))

    forces_t, e_out = pl.pallas_call(
        d_body,
        out_shape=(jax.ShapeDtypeStruct((3, NPAD), f32),
                   jax.ShapeDtypeStruct((1, 1), f32)),
    )(fx_p.reshape(_NC, NPAD), fy_p.reshape(_NC, NPAD),
      fz_p.reshape(_NC, NPAD), fsum.reshape(4, _NW * _L // 4),
      psum.reshape(4, _NW * _L // 4))

    energy = e_out[0, 0]
    forces = forces_t[:, :N].T
    return energy, forces
